# Initial kernel scaffold; baseline (speedup 1.0000x reference)
import jax
import jax.numpy as jnp
from jax.experimental import pallas as pl


def kernel(embedding, table, W, b):
    raise NotImplementedError("write your pallas kernel here")


# speedup vs baseline: 62.6561x; 62.6561x over previous
import jax
import jax.numpy as jnp
from jax import lax
from jax.experimental import pallas as pl
from jax.experimental.pallas import tpu as pltpu
from jax.experimental.pallas import tpu_sc as plsc

VOCAB = 100000
EMBED_DIM = 64
OUT_DIM = 2
BATCH = 4096
SEQ = 200
VPAD = 102400
HALF = VPAD // 2
TS = SEQ // 8
TB = BATCH // 128

_GRID = 10
_BW = HALF // _GRID


def _project_body(t1_ref, t2_ref, w_ref, b_ref, q0_ref, q1_ref):
    pid = pl.program_id(0)
    tbl = jnp.concatenate([t1_ref[...], t2_ref[...]], axis=1)
    q = lax.dot_general(w_ref[...], tbl, (((1,), (0,)), ((), ())),
                        preferred_element_type=jnp.float32)
    ids = lax.broadcasted_iota(jnp.int32, q.shape, 1)
    vids = jnp.where(ids < _BW, ids + pid * _BW, ids - _BW + pid * _BW + HALF)
    q = jnp.where(vids == 0, 0.0, q)
    q = (q + b_ref[...]) * (1.0 / SEQ)
    q = jnp.where(vids >= VOCAB, 0.0, q)
    u16 = lax.bitcast_convert_type(q.astype(jnp.bfloat16), jnp.uint16)
    u = u16.astype(jnp.int32)
    packed = u[:, :_BW] | (u[:, _BW:] << 16)
    q0_ref[...] = packed[0]
    q1_ref[...] = packed[1]


def _project(table_t, W, b2):
    return pl.pallas_call(
        _project_body,
        grid=(_GRID,),
        in_specs=[
            pl.BlockSpec((EMBED_DIM, _BW), lambda i: (0, i)),
            pl.BlockSpec((EMBED_DIM, _BW), lambda i: (0, i + _GRID)),
            pl.BlockSpec((OUT_DIM, EMBED_DIM), lambda i: (0, 0)),
            pl.BlockSpec((OUT_DIM, 1), lambda i: (0, 0)),
        ],
        out_specs=[
            pl.BlockSpec((_BW,), lambda i: (i,)),
            pl.BlockSpec((_BW,), lambda i: (i,)),
        ],
        out_shape=[
            jax.ShapeDtypeStruct((HALF,), jnp.int32),
            jax.ShapeDtypeStruct((HALF,), jnp.int32),
        ],
    )(table_t, table_t, W, b2)


def _sc_gather(q0_hbm, q1_hbm, idx_hbm, out_hbm, q_v, idx_v, out_v,
               sem_i0, sem_i1):
    c = lax.axis_index("c")
    s = lax.axis_index("s")
    cps = [pltpu.async_copy(idx_hbm.at[:, 2 * s + tb], idx_v.at[tb], sem)
           for tb, sem in ((0, sem_i0), (1, sem_i1))]
    for _c, q_hbm in ((0, q0_hbm), (1, q1_hbm)):
        @pl.when(c == _c)
        def _(q_hbm=q_hbm):
            pltpu.sync_copy(q_hbm, q_v)
    for tb in range(2):
        cps[tb].wait()
        for g in range(8):
            def body(ts, acc, _tb=tb, _g=g):
                a = acc
                for si in range(8):
                    v = idx_v[_tb, ts, si, pl.ds(_g * 16, 16)]
                    hi = v >= HALF
                    w = jnp.where(hi, v - HALF, v)
                    u = plsc.load_gather(q_v, [w])
                    bits = jnp.where(hi, u & -65536, u << 16)
                    a = a + plsc.bitcast(bits, jnp.float32)
                return a
            acc = lax.fori_loop(0, TS, body, jnp.zeros((16,), jnp.float32))
            out_v[pl.ds(tb * 128 + g * 16, 16)] = acc
    pltpu.sync_copy(out_v, out_hbm.at[c, pl.ds(s * 256, 256)])


def kernel(embedding, table, W, b):
    q0, q1 = _project(table.T, W, b.reshape(OUT_DIM, 1))
    idx4 = embedding.T.reshape(TS, 8, TB, 128).transpose(0, 2, 1, 3)

    mesh = plsc.VectorSubcoreMesh(core_axis_name="c", subcore_axis_name="s")
    sc = pl.kernel(
        _sc_gather, mesh=mesh,
        compiler_params=pltpu.CompilerParams(
            needs_layout_passes=False, use_tc_tiling_on_sc=False),
        out_type=jax.ShapeDtypeStruct((OUT_DIM, BATCH), jnp.float32),
        scratch_types=[
            pltpu.VMEM((HALF,), jnp.int32),
            pltpu.VMEM((2, TS, 8, 128), jnp.int32),
            pltpu.VMEM((256,), jnp.float32),
            pltpu.SemaphoreType.DMA,
            pltpu.SemaphoreType.DMA,
        ],
    )
    out_t = sc(q0, q1, idx4)
    return out_t.T
